# Initial kernel scaffold; baseline (speedup 1.0000x reference)
import jax
import jax.numpy as jnp
from jax.experimental import pallas as pl


def kernel(var_node_features, con_node_features, node_types, assoc_var, assoc_con, edge_index, edge_types, edge_features, rhs, params):
    raise NotImplementedError("write your pallas kernel here")


# speedup vs baseline: 46.0515x; 46.0515x over previous
import functools

import jax
import jax.numpy as jnp
from jax import lax
from jax.experimental import pallas as pl
from jax.experimental.pallas import tpu as pltpu
from jax.experimental.pallas import tpu_sc as plsc

D32 = 32
NVK, NCK = 30000, 20000
NK = NVK + NCK
EK = 800000
NPAD = 50176
NW = 32
NPS = NPAD // 16
EPW = 25088
ECH = 196
CW = 128
EPAD = NW * EPW
GRID = NPAD // 128

_MESH = dict(core_axis_name="c", subcore_axis_name="s")


def _sc_gather(T, D, K, C):
    @functools.partial(
        pl.kernel,
        out_type=jax.ShapeDtypeStruct((NW * K * C, D), jnp.float32),
        mesh=plsc.VectorSubcoreMesh(**_MESH),
        compiler_params=pltpu.CompilerParams(use_tc_tiling_on_sc=False),
        scratch_types=[pltpu.VMEM((K, C), jnp.int32),
                       pltpu.VMEM((C, D), jnp.float32),
                       pltpu.SemaphoreType.DMA],
    )
    def k(tab, idx, out, idx_v, row_v, sem):
        w = lax.axis_index("c") * 16 + lax.axis_index("s")
        pltpu.sync_copy(idx.at[w], idx_v)

        def body(j, carry):
            pltpu.async_copy(tab.at[idx_v.at[j]], row_v, sem).wait()
            pltpu.sync_copy(row_v, out.at[pl.ds(w * K * C + j * C, C)])
            return carry

        lax.fori_loop(0, K, body, 0)
    return k


def _sc_counts():
    @functools.partial(
        pl.kernel,
        out_type=jax.ShapeDtypeStruct((2, NPAD, 16), jnp.float32),
        mesh=plsc.VectorSubcoreMesh(**_MESH),
        compiler_params=pltpu.CompilerParams(use_tc_tiling_on_sc=False),
        scratch_types=[pltpu.VMEM((ECH, CW), jnp.int32),
                       pltpu.VMEM((CW, 16), jnp.float32),
                       pltpu.VMEM_SHARED((NPAD, 16), jnp.float32)],
    )
    def k(dst, ones_h, zz, out, dst_v, ones_v, c_sh):
        c = lax.axis_index("c")
        s = lax.axis_index("s")
        w = c * 16 + s
        pltpu.sync_copy(zz.at[pl.ds(s * NPS, NPS)], c_sh.at[pl.ds(s * NPS, NPS)])
        pltpu.sync_copy(dst.at[w], dst_v)
        pltpu.sync_copy(ones_h, ones_v)
        plsc.subcore_barrier()

        def body(j, carry):
            pltpu.sync_copy(ones_v, c_sh.at[dst_v.at[j]], add=True)
            return carry

        lax.fori_loop(0, ECH, body, 0)
        plsc.subcore_barrier()
        pltpu.sync_copy(c_sh.at[pl.ds(s * NPS, NPS)],
                        out.at[c, pl.ds(s * NPS, NPS)])
    return k


def _sc_edge():
    @functools.partial(
        pl.kernel,
        out_type=jax.ShapeDtypeStruct((2, NPAD, D32), jnp.float32),
        mesh=plsc.VectorSubcoreMesh(**_MESH),
        compiler_params=pltpu.CompilerParams(use_tc_tiling_on_sc=False),
        scratch_types=[pltpu.VMEM((28, CW), jnp.int32),
                       pltpu.VMEM((28, CW), jnp.int32),
                       pltpu.VMEM((4, CW, D32), jnp.float32),
                       pltpu.VMEM_SHARED((NPAD, D32), jnp.float32),
                       pltpu.SemaphoreType.DMA((4,))],
    )
    def k(M, ptr, dst, zz, out, ptr_v, dst_v, rbuf, s_sh, sems):
        c = lax.axis_index("c")
        s = lax.axis_index("s")
        w = c * 16 + s
        pltpu.sync_copy(zz.at[pl.ds(s * NPS, NPS)], s_sh.at[pl.ds(s * NPS, NPS)])
        plsc.subcore_barrier()

        def outer(g, carry):
            pltpu.sync_copy(ptr.at[w, pl.ds(g * 28, 28)], ptr_v)
            pltpu.sync_copy(dst.at[w, pl.ds(g * 28, 28)], dst_v)

            def body(j4, carry2):
                descs = []
                for b in range(4):
                    j = j4 * 4 + b
                    descs.append(
                        pltpu.async_copy(M.at[ptr_v.at[j]], rbuf.at[b],
                                         sems.at[b]))
                for b in range(4):
                    j = j4 * 4 + b
                    descs[b].wait()
                    pltpu.sync_copy(rbuf.at[b], s_sh.at[dst_v.at[j]], add=True)
                return carry2

            lax.fori_loop(0, 7, body, 0)
            return carry

        lax.fori_loop(0, ECH // 28, outer, 0)
        plsc.subcore_barrier()
        pltpu.sync_copy(s_sh.at[pl.ds(s * NPS, NPS)],
                        out.at[c, pl.ds(s * NPS, NPS)])
    return k


def _row_spec(w=D32):
    return pl.BlockSpec((128, w), lambda i: (i, 0))


def _full_spec(r, w):
    return pl.BlockSpec((r, w), lambda i: (0, 0))


def _ne_body(nf, vW1, vW2, cW1, cW2, o):
    i = pl.program_id(0)
    x = nf[...]
    hv = jnp.maximum(x @ vW1[...], 0.0) @ vW2[...]
    hc = jnp.maximum(x @ cW1[...], 0.0) @ cW2[...]
    rows = lax.broadcasted_iota(jnp.int32, (128, D32), 0) + i * 128
    h = jnp.where(rows < NVK, hv, hc)
    o[...] = jnp.concatenate([h[:, :D32 - 3], x[:, :3]], axis=-1)


def _p_body(x_r, ef_r, W1, W2c, wc, wv, b1, o):
    x = x_r[...]
    ef = ef_r[...]
    h = jax.nn.sigmoid(x @ W1[...] + b1[0:1, :])
    q = h @ W2c[...]
    m0 = x @ wc[...] + q * ef
    c31 = (lax.broadcasted_iota(jnp.int32, (128, D32), 1) == D32 - 1)
    m1 = x @ wv[...] + jnp.where(c31, ef * x[:, D32 - 1:D32], 0.0)
    o[0] = m0
    o[1] = m1


def _q_body(s2, c2, keep, add, o):
    s = s2[0] + s2[1]
    cnt = (c2[0] + c2[1])[:, 0:1]
    rinv = 1.0 / jnp.maximum(cnt, 1.0)
    o[...] = jnp.maximum(keep[...] * s * rinv + add[...], 0.0)


def _f_body(x0, x1, x2, x3, x4, W1, b1, W2, b2, W3, b3, W4, b4, o):
    cat = jnp.concatenate([x0[...], x1[...], x2[...], x3[...], x4[...]], -1)
    h = jnp.maximum(cat @ W1[...] + b1[0:1, :], 0.0)
    h = jnp.maximum(h @ W2[...] + b2[0:1, :], 0.0)
    h = jnp.maximum(h @ W3[...] + b3[0:1, :], 0.0)
    y = h @ W4[...] + b4[0:1, :]
    o[...] = jnp.broadcast_to(y[:, 0:1], (128, 16))


def _tc(body, nout_spec, out_shape, in_specs, args):
    return pl.pallas_call(
        body, grid=(GRID,), in_specs=in_specs, out_specs=nout_spec,
        out_shape=out_shape)(*args)


def _pad_w(W, r=D32, c=D32):
    out = jnp.zeros((r, c), jnp.float32)
    return out.at[:W.shape[0], :W.shape[1]].set(W)


def _brow(b):
    v = jnp.zeros((D32,), jnp.float32).at[:b.shape[0]].set(b)
    return jnp.broadcast_to(v[None, :], (8, D32))


def kernel(var_node_features, con_node_features, node_types, assoc_var,
           assoc_con, edge_index, edge_types, edge_features, rhs, params):
    f32 = jnp.float32
    av = assoc_var.astype(jnp.int32)
    ac = assoc_con.astype(jnp.int32)
    src = edge_index[0].astype(jnp.int32)
    dst = edge_index[1].astype(jnp.int32)
    typ = edge_types.astype(jnp.int32)
    ef = edge_features[:, 0].astype(f32)

    nf = jnp.zeros((NPAD, D32), f32)
    nf = nf.at[:NVK, 0:2].set(var_node_features)
    nf = nf.at[NVK:NK, 0:2].set(con_node_features)
    nf = nf.at[:NK, 2].set(1.0)

    arange_v = jnp.arange(NVK, dtype=jnp.int32)
    arange_c = jnp.arange(NCK, dtype=jnp.int32)
    src_sel = jnp.full((NPAD,), NK, jnp.int32)
    src_sel = src_sel.at[av].set(arange_v)
    src_sel = src_sel.at[ac].set(NVK + arange_c)
    in_var = jnp.zeros((NK,), bool).at[av].set(True)
    in_con = jnp.zeros((NK,), bool).at[ac].set(True)
    rhs_full = jnp.zeros((NK,), f32).at[ac].set(rhs.astype(f32))
    keep = (in_var | in_con).astype(f32)
    sub = rhs_full * (in_con & ~in_var).astype(f32)
    keep_mat = jnp.zeros((NPAD, D32), f32).at[:NK, :].set(
        jnp.broadcast_to(keep[:, None], (NK, D32)))
    e31 = (jnp.arange(D32) == D32 - 1).astype(f32)
    ef_mat = jnp.zeros((NPAD, D32), f32).at[:NK, :].set(
        jnp.broadcast_to(ef[:NK, None], (NK, D32)))

    npad_e = EPAD - EK
    fill = jnp.arange(npad_e, dtype=jnp.int32)
    ptr = jnp.concatenate([src + typ * NPAD, fill % 4096])
    dste = jnp.concatenate([dst, NK + (fill % 128)])
    ptr = ptr.reshape(NW, ECH, CW)
    dste = dste.reshape(NW, ECH, CW)

    z32 = jnp.zeros((NPAD, D32), f32)
    z16 = jnp.zeros((NPAD, 16), f32)
    ones16 = jnp.ones((CW, 16), f32)

    def aug2(p):
        W1a = jnp.zeros((D32, D32), f32)
        W1a = W1a.at[0:2, :D32 - 3].set(p['W1'])
        W1a = W1a.at[2, :D32 - 3].set(p['b1'])
        W1a = W1a.at[2, D32 - 1].set(1.0)
        W2a = jnp.zeros((D32, D32), f32)
        W2a = W2a.at[:D32 - 3, :D32 - 3].set(p['W2'])
        W2a = W2a.at[D32 - 1, :D32 - 3].set(p['b2'])
        return W1a, W2a

    vW1, vW2 = aug2(params['var_mlp'])
    cW1, cW2 = aug2(params['con_mlp'])
    wspec = _full_spec(D32, D32)
    ne = _tc(_ne_body, _row_spec(), jax.ShapeDtypeStruct((NPAD, D32), f32),
             [_row_spec(), wspec, wspec, wspec, wspec],
             (nf, vW1, vW2, cW1, cW2))

    x0 = _sc_gather(NPAD, D32, 14, 112)(ne, src_sel.reshape(NW, 14, 112))

    cnt2 = _sc_counts()(dste, ones16, z16)

    edge_k = _sc_edge()
    x = x0
    xs = [x0]
    for name in ('conv1', 'conv2', 'conv3', 'conv4'):
        p = params[name]
        W1p = _pad_w(p['h2v_W1'])
        W2c = jnp.zeros((D32, D32), f32)
        W2c = W2c.at[:D32 - 1, D32 - 1].set(p['h2v_W2'][:, 0])
        W2c = W2c.at[D32 - 1, D32 - 1].set(2.0 * p['h2v_b2'][0])
        wc = _pad_w(p['w_cons'])
        wv = _pad_w(p['w_vars'])
        b1r = _brow(p['h2v_b1'])
        add_mat = jnp.zeros((NPAD, D32), f32).at[:NK, :].set(
            p['bias'][None, :] - sub[:, None] * e31[None, :])

        M = _tc(_p_body, pl.BlockSpec((2, 128, D32), lambda i: (0, i, 0)),
                jax.ShapeDtypeStruct((2, NPAD, D32), f32),
                [_row_spec(), _row_spec(), wspec, wspec, wspec, wspec,
                 _full_spec(8, D32)],
                (x, ef_mat, W1p, W2c, wc, wv, b1r))
        M2 = M.reshape(2 * NPAD, D32)

        s2 = edge_k(M2, ptr, dste, z32)

        x = _tc(_q_body, _row_spec(), jax.ShapeDtypeStruct((NPAD, D32), f32),
                [pl.BlockSpec((2, 128, D32), lambda i: (0, i, 0)),
                 pl.BlockSpec((2, 128, 16), lambda i: (0, i, 0)),
                 _row_spec(), _row_spec()],
                (s2, cnt2, keep_mat, add_mat))
        xs.append(x)

    fW1, fb1 = params['fc1']
    fW2, fb2 = params['fc2']
    fW3, fb3 = params['fc3']
    fW4, fb4 = params['fc4']
    W4p = jnp.zeros((D32, D32), f32).at[:, 0:1].set(fW4)
    y16 = _tc(_f_body, pl.BlockSpec((128, 16), lambda i: (i, 0)),
              jax.ShapeDtypeStruct((NPAD, 16), f32),
              [_row_spec()] * 5 +
              [_full_spec(5 * D32, D32), _full_spec(8, D32), wspec,
               _full_spec(8, D32), wspec, _full_spec(8, D32), wspec,
               _full_spec(8, D32)],
              (*xs, fW1, _brow(fb1), _pad_w(fW2), _brow(fb2), _pad_w(fW3),
               _brow(fb3), W4p, jnp.full((8, D32), fb4[0], f32)))

    gpad = 30720 - NVK
    gidx = jnp.concatenate([av, jnp.arange(gpad, dtype=jnp.int32) % 8192])
    yg = _sc_gather(NPAD, 16, 10, 96)(y16, gidx.reshape(NW, 10, 96))
    return yg[:NVK, 0]


# speedup vs baseline: 144.7192x; 3.1426x over previous
import functools

import jax
import jax.numpy as jnp
from jax import lax
from jax.experimental import pallas as pl
from jax.experimental.pallas import tpu as pltpu
from jax.experimental.pallas import tpu_sc as plsc

D32 = 32
NVK, NCK = 30000, 20000
NK = NVK + NCK
EK = 800000
NPAD = 50176
NR = NPAD // 4
BR = 256
G2 = NR // BR
NW = 32
NPS = NPAD // 16
EPW = 25088
ECH = 196
CW = 128
EPAD = NW * EPW

_MESH = dict(core_axis_name="c", subcore_axis_name="s")


def _sc_gather(T, D, K, C):
    @functools.partial(
        pl.kernel,
        out_type=jax.ShapeDtypeStruct((NW * K * C, D), jnp.float32),
        mesh=plsc.VectorSubcoreMesh(**_MESH),
        compiler_params=pltpu.CompilerParams(use_tc_tiling_on_sc=False),
        scratch_types=[pltpu.VMEM((K, C), jnp.int32),
                       pltpu.VMEM((C, D), jnp.float32),
                       pltpu.SemaphoreType.DMA],
    )
    def k(tab, idx, out, idx_v, row_v, sem):
        w = lax.axis_index("c") * 16 + lax.axis_index("s")
        pltpu.sync_copy(idx.at[w], idx_v)

        def body(j, carry):
            pltpu.async_copy(tab.at[idx_v.at[j]], row_v, sem).wait()
            pltpu.sync_copy(row_v, out.at[pl.ds(w * K * C + j * C, C)])
            return carry

        lax.fori_loop(0, K, body, 0)
    return k


def _sc_counts():
    @functools.partial(
        pl.kernel,
        out_type=jax.ShapeDtypeStruct((2, NPAD, D32), jnp.float32),
        mesh=plsc.VectorSubcoreMesh(**_MESH),
        compiler_params=pltpu.CompilerParams(use_tc_tiling_on_sc=False),
        scratch_types=[pltpu.VMEM((28, CW), jnp.int32),
                       pltpu.VMEM((CW, D32), jnp.float32),
                       pltpu.VMEM_SHARED((NPAD, D32), jnp.float32)],
    )
    def k(dst, ones_h, zz, out, dst_v, ones_v, c_sh):
        c = lax.axis_index("c")
        s = lax.axis_index("s")
        w = c * 16 + s
        pltpu.sync_copy(zz.at[pl.ds(s * NPS, NPS)], c_sh.at[pl.ds(s * NPS, NPS)])
        pltpu.sync_copy(ones_h, ones_v)
        plsc.subcore_barrier()

        def outer(g, carry):
            pltpu.sync_copy(dst.at[w, pl.ds(g * 28, 28)], dst_v)

            def body(j, carry2):
                pltpu.sync_copy(ones_v, c_sh.at[dst_v.at[j]], add=True)
                return carry2

            lax.fori_loop(0, 28, body, 0)
            return carry

        lax.fori_loop(0, ECH // 28, outer, 0)
        plsc.subcore_barrier()
        pltpu.sync_copy(c_sh.at[pl.ds(s * NPS, NPS)],
                        out.at[c, pl.ds(s * NPS, NPS)])
    return k


def _sc_edge():
    @functools.partial(
        pl.kernel,
        out_type=jax.ShapeDtypeStruct((2, NPAD, D32), jnp.float32),
        mesh=plsc.VectorSubcoreMesh(**_MESH),
        compiler_params=pltpu.CompilerParams(use_tc_tiling_on_sc=False),
        scratch_types=[pltpu.VMEM((28, CW), jnp.int32),
                       pltpu.VMEM((28, CW), jnp.int32),
                       pltpu.VMEM((4, CW, D32), jnp.float32),
                       pltpu.VMEM_SHARED((NPAD, D32), jnp.float32),
                       pltpu.SemaphoreType.DMA((4,))],
    )
    def k(M, ptr, dst, zz, out, ptr_v, dst_v, rbuf, s_sh, sems):
        c = lax.axis_index("c")
        s = lax.axis_index("s")
        w = c * 16 + s
        pltpu.sync_copy(zz.at[pl.ds(s * NPS, NPS)], s_sh.at[pl.ds(s * NPS, NPS)])
        plsc.subcore_barrier()

        def outer(g, carry):
            pltpu.sync_copy(ptr.at[w, pl.ds(g * 28, 28)], ptr_v)
            pltpu.sync_copy(dst.at[w, pl.ds(g * 28, 28)], dst_v)

            def body(j4, carry2):
                descs = []
                for b in range(4):
                    j = j4 * 4 + b
                    descs.append(
                        pltpu.async_copy(M.at[ptr_v.at[j]], rbuf.at[b],
                                         sems.at[b]))
                for b in range(4):
                    j = j4 * 4 + b
                    descs[b].wait()
                    pltpu.sync_copy(rbuf.at[b], s_sh.at[dst_v.at[j]], add=True)
                return carry2

            lax.fori_loop(0, 7, body, 0)
            return carry

        lax.fori_loop(0, ECH // 28, outer, 0)
        plsc.subcore_barrier()
        pltpu.sync_copy(s_sh.at[pl.ds(s * NPS, NPS)],
                        out.at[c, pl.ds(s * NPS, NPS)])
    return k


def _rspec():
    return pl.BlockSpec((BR, 128), lambda i: (i, 0))


def _sspec():
    return pl.BlockSpec((2, BR, 128), lambda i: (0, i, 0))


def _wspec():
    return pl.BlockSpec((128, 128), lambda i: (0, 0))


def _bspec():
    return pl.BlockSpec((8, 128), lambda i: (0, 0))


def _lane31(shape):
    return (lax.broadcasted_iota(jnp.int32, shape, 1) % D32) == D32 - 1


def _ne_body(nf, vW1, vW2, cW1, cW2, SH, o):
    i = pl.program_id(0)
    x = nf[...]
    hv = jnp.maximum(x @ vW1[...], 0.0) @ vW2[...]
    hc = jnp.maximum(x @ cW1[...], 0.0) @ cW2[...]
    vrow = lax.broadcasted_iota(jnp.int32, (BR, 128), 0) + i * BR
    h = jnp.where(vrow < NVK // 4, hv, hc)
    lane = lax.broadcasted_iota(jnp.int32, (BR, 128), 1) % D32
    o[...] = jnp.where(lane < D32 - 3, h, 0.0) + x @ SH[...]


def _mtabs(x, ef, W1, W2c, wc, wv, B31, b1):
    h = jax.nn.sigmoid(x @ W1[...] + b1[0:1, :])
    q = h @ W2c[...]
    m0 = x @ wc[...] + q * ef
    viol = (x @ B31[...]) * ef
    m1 = x @ wv[...] + jnp.where(_lane31(viol.shape), viol, 0.0)
    return m0, m1


def _p1_body(x_r, ef_r, W1, W2c, wc, wv, B31, b1, oM):
    m0, m1 = _mtabs(x_r[...], ef_r[...], W1, W2c, wc, wv, B31, b1)
    oM[0] = m0
    oM[1] = m1


def _qp_body(s2, krinv, add, ef_r, W1, W2c, wc, wv, B31, b1, ox, oM):
    x = jnp.maximum((s2[0] + s2[1]) * krinv[...] + add[...], 0.0)
    m0, m1 = _mtabs(x, ef_r[...], W1, W2c, wc, wv, B31, b1)
    ox[...] = x
    oM[0] = m0
    oM[1] = m1


def _ri_body(c2, keep, o):
    o[...] = keep[...] / jnp.maximum(c2[0] + c2[1], 1.0)


def _f_body(s2, krinv, add, x0, x1, x2, x3,
            Wa, Wb, Wc, Wd, We, b1, W2, b2, W3, b3, W4, b4, o):
    x4 = jnp.maximum((s2[0] + s2[1]) * krinv[...] + add[...], 0.0)
    h = (x0[...] @ Wa[...] + x1[...] @ Wb[...] + x2[...] @ Wc[...] +
         x3[...] @ Wd[...] + x4 @ We[...])
    h = jnp.maximum(h + b1[0:1, :], 0.0)
    h = jnp.maximum(h @ W2[...] + b2[0:1, :], 0.0)
    h = jnp.maximum(h @ W3[...] + b3[0:1, :], 0.0)
    o[...] = h @ W4[...] + b4[0:1, :]


def _tc(body, out_specs, out_shape, in_specs, args):
    return pl.pallas_call(
        body, grid=(G2,), in_specs=in_specs, out_specs=out_specs,
        out_shape=out_shape)(*args)


_EYE4 = None


def _bd(W):
    return jnp.kron(jnp.eye(4, dtype=jnp.float32), W)


def _pad_w(W, r=D32, c=D32):
    out = jnp.zeros((r, c), jnp.float32)
    return out.at[:W.shape[0], :W.shape[1]].set(W)


def _btile(b):
    v = jnp.zeros((D32,), jnp.float32).at[:b.shape[0]].set(b)
    return jnp.broadcast_to(jnp.tile(v, 4)[None, :], (8, 128))


def kernel(var_node_features, con_node_features, node_types, assoc_var,
           assoc_con, edge_index, edge_types, edge_features, rhs, params):
    f32 = jnp.float32
    av = assoc_var.astype(jnp.int32)
    ac = assoc_con.astype(jnp.int32)
    src = edge_index[0].astype(jnp.int32)
    dst = edge_index[1].astype(jnp.int32)
    typ = edge_types.astype(jnp.int32)
    ef = edge_features[:, 0].astype(f32)

    nf = jnp.zeros((NPAD, D32), f32)
    nf = nf.at[:NVK, 0:2].set(var_node_features)
    nf = nf.at[NVK:NK, 0:2].set(con_node_features)
    nf = nf.at[:NK, 2].set(1.0)
    nf4 = nf.reshape(NR, 128)

    va_upd = jnp.stack([jnp.arange(NVK, dtype=f32), jnp.ones((NVK,), f32)], -1)
    ca_upd = jnp.stack([NVK + jnp.arange(NCK, dtype=f32),
                        jnp.ones((NCK,), f32), rhs.astype(f32)], -1)
    tva = jnp.zeros((NK, 2), f32).at[av].set(va_upd)
    tca = jnp.zeros((NK, 3), f32).at[ac].set(ca_upd)
    in_var = tva[:, 1] > 0.5
    in_con = tca[:, 1] > 0.5
    src_sel = jnp.where(in_con, tca[:, 0], jnp.where(in_var, tva[:, 0], NK))
    src_sel = jnp.concatenate([src_sel.astype(jnp.int32),
                               jnp.full((NPAD - NK,), NK, jnp.int32)])
    keep = (in_var | in_con).astype(f32)
    sub = tca[:, 2] * (in_con & ~in_var).astype(f32)
    keep_mat = jnp.zeros((NPAD, D32), f32).at[:NK, :].set(
        jnp.broadcast_to(keep[:, None], (NK, D32)))
    e31 = (jnp.arange(D32) == D32 - 1).astype(f32)
    ef_mat = jnp.zeros((NPAD, D32), f32).at[:NK, :].set(
        jnp.broadcast_to(ef[:NK, None], (NK, D32)))
    keep4 = keep_mat.reshape(NR, 128)
    ef4 = ef_mat.reshape(NR, 128)

    npad_e = EPAD - EK
    fill = jnp.arange(npad_e, dtype=jnp.int32)
    ptr = jnp.concatenate([src + typ * NPAD, fill % 4096]).reshape(NW, ECH, CW)
    dste = jnp.concatenate([dst, NK + (fill % 128)]).reshape(NW, ECH, CW)

    z32 = jnp.zeros((NPAD, D32), f32)
    ones32 = jnp.ones((CW, D32), f32)

    def aug2(p):
        W1a = jnp.zeros((D32, D32), f32)
        W1a = W1a.at[0:2, :D32 - 3].set(p['W1'])
        W1a = W1a.at[2, :D32 - 3].set(p['b1'])
        W1a = W1a.at[2, D32 - 1].set(1.0)
        W2a = jnp.zeros((D32, D32), f32)
        W2a = W2a.at[:D32 - 3, :D32 - 3].set(p['W2'])
        W2a = W2a.at[D32 - 1, :D32 - 3].set(p['b2'])
        return _bd(W1a), _bd(W2a)

    vW1, vW2 = aug2(params['var_mlp'])
    cW1, cW2 = aug2(params['con_mlp'])
    SH = jnp.zeros((D32, D32), f32)
    SH = SH.at[0, D32 - 3].set(1.0).at[1, D32 - 2].set(1.0).at[2, D32 - 1].set(1.0)
    SHb = _bd(SH)
    B31b = _bd(jnp.zeros((D32, D32), f32).at[D32 - 1, :].set(1.0))

    ne4 = _tc(_ne_body, _rspec(), jax.ShapeDtypeStruct((NR, 128), f32),
              [_rspec()] + [_wspec()] * 5,
              (nf4, vW1, vW2, cW1, cW2, SHb))
    ne = ne4.reshape(NPAD, D32)

    x0 = _sc_gather(NPAD, D32, 14, 112)(ne, src_sel.reshape(NW, 14, 112))
    cnt2 = _sc_counts()(dste, ones32, z32)

    krinv = _tc(_ri_body, _rspec(), jax.ShapeDtypeStruct((NR, 128), f32),
                [_sspec(), _rspec()],
                (cnt2.reshape(2, NR, 128), keep4))

    def conv_w(p):
        W2c = jnp.zeros((D32, D32), f32)
        W2c = W2c.at[:D32 - 1, D32 - 1].set(p['h2v_W2'][:, 0])
        W2c = W2c.at[D32 - 1, D32 - 1].set(2.0 * p['h2v_b2'][0])
        add_mat = jnp.zeros((NPAD, D32), f32).at[:NK, :].set(
            p['bias'][None, :] - sub[:, None] * e31[None, :])
        return (_bd(_pad_w(p['h2v_W1'])), _bd(W2c), _bd(_pad_w(p['w_cons'])),
                _bd(_pad_w(p['w_vars'])), _btile(p['h2v_b1']),
                add_mat.reshape(NR, 128))

    edge_k = _sc_edge()
    wsp = [_wspec()] * 4 + [pl.BlockSpec((128, 128), lambda i: (0, 0))]
    m_struct = jax.ShapeDtypeStruct((2, NR, 128), f32)
    x_struct = jax.ShapeDtypeStruct((NR, 128), f32)

    p1 = conv_w(params['conv1'])
    M = _tc(_p1_body, _sspec(), m_struct,
            [_rspec(), _rspec(), _wspec(), _wspec(), _wspec(), _wspec(),
             _wspec(), _bspec()],
            (x0.reshape(NR, 128), ef4, p1[0], p1[1], p1[2], p1[3],
             _bd(jnp.zeros((D32, D32), f32).at[D32 - 1, :].set(1.0)), p1[4]))
    s2 = edge_k(M.reshape(2 * NPAD, D32), ptr, dste, z32)

    xs4 = [x0.reshape(NR, 128)]
    for name in ('conv2', 'conv3', 'conv4'):
        p = conv_w(params[name])
        add_prev = conv_w(params[{'conv2': 'conv1', 'conv3': 'conv2',
                                  'conv4': 'conv3'}[name]])[5]
        x, M = _tc(_qp_body, [_rspec(), _sspec()], (x_struct, m_struct),
                   [_sspec(), _rspec(), _rspec(), _rspec(), _wspec(),
                    _wspec(), _wspec(), _wspec(), _wspec(), _bspec()],
                   (s2.reshape(2, NR, 128), krinv, add_prev, ef4,
                    p[0], p[1], p[2], p[3], B31b, p[4]))
        xs4.append(x)
        s2 = edge_k(M.reshape(2 * NPAD, D32), ptr, dste, z32)

    add4 = conv_w(params['conv4'])[5]
    fW1, fb1 = params['fc1']
    fW2, fb2 = params['fc2']
    fW3, fb3 = params['fc3']
    fW4, fb4 = params['fc4']
    Wk = [_bd(fW1[32 * k:32 * (k + 1), :]) for k in range(5)]
    W4b = _bd(jnp.broadcast_to(fW4, (D32, D32)))
    b4t = jnp.full((8, 128), fb4[0], f32)

    y4 = _tc(_f_body, _rspec(), x_struct,
             [_sspec()] + [_rspec()] * 6 + [_wspec()] * 5 +
             [_bspec(), _wspec(), _bspec(), _wspec(), _bspec(), _wspec(),
              _bspec()],
             (s2.reshape(2, NR, 128), krinv, add4, xs4[0], xs4[1], xs4[2],
              xs4[3], Wk[0], Wk[1], Wk[2], Wk[3], Wk[4], _btile(fb1),
              _bd(_pad_w(fW2)), _btile(fb2), _bd(_pad_w(fW3)), _btile(fb3),
              W4b, b4t))

    gpad = 30720 - NVK
    gidx = jnp.concatenate([av, jnp.arange(gpad, dtype=jnp.int32) % 8192])
    yg = _sc_gather(NPAD, D32, 10, 96)(y4.reshape(NPAD, D32),
                                       gidx.reshape(NW, 10, 96))
    return yg[:NVK, 0]


# speedup vs baseline: 148.7834x; 1.0281x over previous
import functools

import jax
import jax.numpy as jnp
from jax import lax
from jax.experimental import pallas as pl
from jax.experimental.pallas import tpu as pltpu
from jax.experimental.pallas import tpu_sc as plsc

D32 = 32
NVK, NCK = 30000, 20000
NK = NVK + NCK
EK = 800000
NPAD = 50176
NR = NPAD // 4
BR = 256
G2 = NR // BR
NW = 32
NPS = NPAD // 16
EPW = 25088
ECH = 196
CW = 128
EPAD = NW * EPW

_MESH = dict(core_axis_name="c", subcore_axis_name="s")


def _sc_gather(T, D, K, C):
    @functools.partial(
        pl.kernel,
        out_type=jax.ShapeDtypeStruct((NW * K * C, D), jnp.float32),
        mesh=plsc.VectorSubcoreMesh(**_MESH),
        compiler_params=pltpu.CompilerParams(use_tc_tiling_on_sc=False),
        scratch_types=[pltpu.VMEM((K, C), jnp.int32),
                       pltpu.VMEM((C, D), jnp.float32),
                       pltpu.SemaphoreType.DMA],
    )
    def k(tab, idx, out, idx_v, row_v, sem):
        w = lax.axis_index("c") * 16 + lax.axis_index("s")
        pltpu.sync_copy(idx.at[w], idx_v)

        def body(j, carry):
            pltpu.async_copy(tab.at[idx_v.at[j]], row_v, sem).wait()
            pltpu.sync_copy(row_v, out.at[pl.ds(w * K * C + j * C, C)])
            return carry

        lax.fori_loop(0, K, body, 0)
    return k


def _sc_counts():
    @functools.partial(
        pl.kernel,
        out_type=jax.ShapeDtypeStruct((2, NPAD, D32), jnp.float32),
        mesh=plsc.VectorSubcoreMesh(**_MESH),
        compiler_params=pltpu.CompilerParams(use_tc_tiling_on_sc=False),
        scratch_types=[pltpu.VMEM((28, CW), jnp.int32),
                       pltpu.VMEM((CW, D32), jnp.float32),
                       pltpu.VMEM_SHARED((NPAD, D32), jnp.float32)],
    )
    def k(dst, ones_h, zz, out, dst_v, ones_v, c_sh):
        c = lax.axis_index("c")
        s = lax.axis_index("s")
        w = c * 16 + s
        pltpu.sync_copy(zz.at[pl.ds(s * NPS, NPS)], c_sh.at[pl.ds(s * NPS, NPS)])
        pltpu.sync_copy(ones_h, ones_v)
        plsc.subcore_barrier()

        def outer(g, carry):
            pltpu.sync_copy(dst.at[w, pl.ds(g * 28, 28)], dst_v)

            def body(j, carry2):
                pltpu.sync_copy(ones_v, c_sh.at[dst_v.at[j]], add=True)
                return carry2

            lax.fori_loop(0, 28, body, 0)
            return carry

        lax.fori_loop(0, ECH // 28, outer, 0)
        plsc.subcore_barrier()
        pltpu.sync_copy(c_sh.at[pl.ds(s * NPS, NPS)],
                        out.at[c, pl.ds(s * NPS, NPS)])
    return k


def _sc_edge():
    @functools.partial(
        pl.kernel,
        out_type=jax.ShapeDtypeStruct((2, NPAD, D32), jnp.float32),
        mesh=plsc.VectorSubcoreMesh(**_MESH),
        compiler_params=pltpu.CompilerParams(use_tc_tiling_on_sc=False),
        scratch_types=[pltpu.VMEM((28, CW), jnp.int32),
                       pltpu.VMEM((28, CW), jnp.int32),
                       pltpu.VMEM((4, CW, D32), jnp.float32),
                       pltpu.VMEM_SHARED((NPAD, D32), jnp.float32),
                       pltpu.SemaphoreType.DMA((4,))],
    )
    def k(M, ptr, dst, zz, out, ptr_v, dst_v, rbuf, s_sh, sems):
        c = lax.axis_index("c")
        s = lax.axis_index("s")
        w = c * 16 + s
        pltpu.sync_copy(zz.at[pl.ds(s * NPS, NPS)], s_sh.at[pl.ds(s * NPS, NPS)])
        plsc.subcore_barrier()

        def outer(g, carry):
            pltpu.sync_copy(ptr.at[w, pl.ds(g * 28, 28)], ptr_v)
            pltpu.sync_copy(dst.at[w, pl.ds(g * 28, 28)], dst_v)

            def body(j4, carry2):
                descs = []
                for b in range(4):
                    j = j4 * 4 + b
                    descs.append(
                        pltpu.async_copy(M.at[ptr_v.at[j]], rbuf.at[b],
                                         sems.at[b]))
                for b in range(4):
                    j = j4 * 4 + b
                    descs[b].wait()
                    pltpu.sync_copy(rbuf.at[b], s_sh.at[dst_v.at[j]], add=True)
                return carry2

            lax.fori_loop(0, 7, body, 0)
            return carry

        lax.fori_loop(0, ECH // 28, outer, 0)
        plsc.subcore_barrier()
        pltpu.sync_copy(s_sh.at[pl.ds(s * NPS, NPS)],
                        out.at[c, pl.ds(s * NPS, NPS)])
    return k


def _rspec():
    return pl.BlockSpec((BR, 128), lambda i: (i, 0))


def _sspec():
    return pl.BlockSpec((2, BR, 128), lambda i: (0, i, 0))


def _wspec():
    return pl.BlockSpec((128, 128), lambda i: (0, 0))


def _bspec():
    return pl.BlockSpec((8, 128), lambda i: (0, 0))


def _lane31(shape):
    return (lax.broadcasted_iota(jnp.int32, shape, 1) % D32) == D32 - 1


def _ne_body(nf, vW1, vW2, cW1, cW2, SH, o):
    i = pl.program_id(0)
    x = nf[...]
    hv = jnp.maximum(x @ vW1[...], 0.0) @ vW2[...]
    hc = jnp.maximum(x @ cW1[...], 0.0) @ cW2[...]
    vrow = lax.broadcasted_iota(jnp.int32, (BR, 128), 0) + i * BR
    h = jnp.where(vrow < NVK // 4, hv, hc)
    lane = lax.broadcasted_iota(jnp.int32, (BR, 128), 1) % D32
    o[...] = jnp.where(lane < D32 - 3, h, 0.0) + x @ SH[...]


def _mtabs(x, ef, W1, W2c, wc, wv, B31, b1):
    h = jax.nn.sigmoid(x @ W1[...] + b1[0:1, :])
    q = h @ W2c[...]
    m0 = x @ wc[...] + q * ef
    viol = (x @ B31[...]) * ef
    m1 = x @ wv[...] + jnp.where(_lane31(viol.shape), viol, 0.0)
    return m0, m1


def _p1_body(x_r, ef_r, W1, W2c, wc, wv, B31, b1, oM):
    m0, m1 = _mtabs(x_r[...], ef_r[...], W1, W2c, wc, wv, B31, b1)
    oM[0] = m0
    oM[1] = m1


def _qp_body(s2, krinv, sub, ef_r, W1, W2c, wc, wv, B31, b1, bk, ox, oM):
    x = jnp.maximum((s2[0] + s2[1]) * krinv[...] + bk[0:1, :] - sub[...], 0.0)
    m0, m1 = _mtabs(x, ef_r[...], W1, W2c, wc, wv, B31, b1)
    ox[...] = x
    oM[0] = m0
    oM[1] = m1


def _ri_body(c2, keep, o):
    o[...] = keep[...] / jnp.maximum(c2[0] + c2[1], 1.0)


def _f_body(s2, krinv, sub, x0, x1, x2, x3,
            Wa, Wb, Wc, Wd, We, bk, b1, W2, b2, W3, b3, W4, b4, o):
    x4 = jnp.maximum((s2[0] + s2[1]) * krinv[...] + bk[0:1, :] - sub[...], 0.0)
    h = (x0[...] @ Wa[...] + x1[...] @ Wb[...] + x2[...] @ Wc[...] +
         x3[...] @ Wd[...] + x4 @ We[...])
    h = jnp.maximum(h + b1[0:1, :], 0.0)
    h = jnp.maximum(h @ W2[...] + b2[0:1, :], 0.0)
    h = jnp.maximum(h @ W3[...] + b3[0:1, :], 0.0)
    o[...] = h @ W4[...] + b4[0:1, :]


def _tc(body, out_specs, out_shape, in_specs, args):
    return pl.pallas_call(
        body, grid=(G2,), in_specs=in_specs, out_specs=out_specs,
        out_shape=out_shape)(*args)


_EYE4 = None


def _bd(W):
    return jnp.kron(jnp.eye(4, dtype=jnp.float32), W)


def _pad_w(W, r=D32, c=D32):
    out = jnp.zeros((r, c), jnp.float32)
    return out.at[:W.shape[0], :W.shape[1]].set(W)


def _btile(b):
    v = jnp.zeros((D32,), jnp.float32).at[:b.shape[0]].set(b)
    return jnp.broadcast_to(jnp.tile(v, 4)[None, :], (8, 128))


def kernel(var_node_features, con_node_features, node_types, assoc_var,
           assoc_con, edge_index, edge_types, edge_features, rhs, params):
    f32 = jnp.float32
    av = assoc_var.astype(jnp.int32)
    ac = assoc_con.astype(jnp.int32)
    src = edge_index[0].astype(jnp.int32)
    dst = edge_index[1].astype(jnp.int32)
    typ = edge_types.astype(jnp.int32)
    ef = edge_features[:, 0].astype(f32)

    nf = jnp.zeros((NPAD, D32), f32)
    nf = nf.at[:NVK, 0:2].set(var_node_features)
    nf = nf.at[NVK:NK, 0:2].set(con_node_features)
    nf = nf.at[:NK, 2].set(1.0)
    nf4 = nf.reshape(NR, 128)

    va_upd = jnp.stack([jnp.arange(NVK, dtype=f32), jnp.ones((NVK,), f32)], -1)
    ca_upd = jnp.stack([NVK + jnp.arange(NCK, dtype=f32),
                        jnp.ones((NCK,), f32), rhs.astype(f32)], -1)
    tva = jnp.zeros((NK, 2), f32).at[av].set(va_upd)
    tca = jnp.zeros((NK, 3), f32).at[ac].set(ca_upd)
    in_var = tva[:, 1] > 0.5
    in_con = tca[:, 1] > 0.5
    src_sel = jnp.where(in_con, tca[:, 0], jnp.where(in_var, tva[:, 0], NK))
    src_sel = jnp.concatenate([src_sel.astype(jnp.int32),
                               jnp.full((NPAD - NK,), NK, jnp.int32)])
    keep = (in_var | in_con).astype(f32)
    sub = tca[:, 2] * (in_con & ~in_var).astype(f32)
    keep_mat = jnp.zeros((NPAD, D32), f32).at[:NK, :].set(
        jnp.broadcast_to(keep[:, None], (NK, D32)))
    e31 = (jnp.arange(D32) == D32 - 1).astype(f32)
    ef_mat = jnp.zeros((NPAD, D32), f32).at[:NK, :].set(
        jnp.broadcast_to(ef[:NK, None], (NK, D32)))
    keep4 = keep_mat.reshape(NR, 128)
    ef4 = ef_mat.reshape(NR, 128)
    sub_mat = jnp.zeros((NPAD, D32), f32).at[:NK, :].set(
        sub[:, None] * e31[None, :])
    sub4 = sub_mat.reshape(NR, 128)
    pad_mat = jnp.zeros((NPAD, D32), f32).at[:NK, :].set(1.0)
    pad4 = pad_mat.reshape(NR, 128)

    npad_e = EPAD - EK
    fill = jnp.arange(npad_e, dtype=jnp.int32)
    ptr = jnp.concatenate([src + typ * NPAD, fill % 4096]).reshape(NW, ECH, CW)
    dste = jnp.concatenate([dst, NK + (fill % 128)]).reshape(NW, ECH, CW)

    z32 = jnp.zeros((NPAD, D32), f32)
    ones32 = jnp.ones((CW, D32), f32)
    cnt2 = _sc_counts()(dste, ones32, z32)

    def aug2(p):
        W1a = jnp.zeros((D32, D32), f32)
        W1a = W1a.at[0:2, :D32 - 3].set(p['W1'])
        W1a = W1a.at[2, :D32 - 3].set(p['b1'])
        W1a = W1a.at[2, D32 - 1].set(1.0)
        W2a = jnp.zeros((D32, D32), f32)
        W2a = W2a.at[:D32 - 3, :D32 - 3].set(p['W2'])
        W2a = W2a.at[D32 - 1, :D32 - 3].set(p['b2'])
        return _bd(W1a), _bd(W2a)

    vW1, vW2 = aug2(params['var_mlp'])
    cW1, cW2 = aug2(params['con_mlp'])
    SH = jnp.zeros((D32, D32), f32)
    SH = SH.at[0, D32 - 3].set(1.0).at[1, D32 - 2].set(1.0).at[2, D32 - 1].set(1.0)
    SHb = _bd(SH)
    B31b = _bd(jnp.zeros((D32, D32), f32).at[D32 - 1, :].set(1.0))

    ne4 = _tc(_ne_body, _rspec(), jax.ShapeDtypeStruct((NR, 128), f32),
              [_rspec()] + [_wspec()] * 5,
              (nf4, vW1, vW2, cW1, cW2, SHb))
    ne = ne4.reshape(NPAD, D32)

    x0 = _sc_gather(NPAD, D32, 14, 112)(ne, src_sel.reshape(NW, 14, 112))

    krinv = _tc(_ri_body, _rspec(), jax.ShapeDtypeStruct((NR, 128), f32),
                [_sspec(), _rspec()],
                (cnt2.reshape(2, NR, 128), keep4))

    def conv_w(p):
        W2c = jnp.zeros((D32, D32), f32)
        W2c = W2c.at[:D32 - 1, D32 - 1].set(p['h2v_W2'][:, 0])
        W2c = W2c.at[D32 - 1, D32 - 1].set(2.0 * p['h2v_b2'][0])
        return (_bd(_pad_w(p['h2v_W1'])), _bd(W2c), _bd(_pad_w(p['w_cons'])),
                _bd(_pad_w(p['w_vars'])), _btile(p['h2v_b1']),
                _btile(p['bias']))

    edge_k = _sc_edge()
    wsp = [_wspec()] * 4 + [pl.BlockSpec((128, 128), lambda i: (0, 0))]
    m_struct = jax.ShapeDtypeStruct((2, NR, 128), f32)
    x_struct = jax.ShapeDtypeStruct((NR, 128), f32)

    p1 = conv_w(params['conv1'])
    M = _tc(_p1_body, _sspec(), m_struct,
            [_rspec(), _rspec(), _wspec(), _wspec(), _wspec(), _wspec(),
             _wspec(), _bspec()],
            (x0.reshape(NR, 128), ef4, p1[0], p1[1], p1[2], p1[3],
             _bd(jnp.zeros((D32, D32), f32).at[D32 - 1, :].set(1.0)), p1[4]))
    s2 = edge_k(M.reshape(2 * NPAD, D32), ptr, dste, z32)

    xs4 = [x0.reshape(NR, 128)]
    for name in ('conv2', 'conv3', 'conv4'):
        p = conv_w(params[name])
        bias_prev = conv_w(params[{'conv2': 'conv1', 'conv3': 'conv2',
                                   'conv4': 'conv3'}[name]])[5]
        x, M = _tc(_qp_body, [_rspec(), _sspec()], (x_struct, m_struct),
                   [_sspec(), _rspec(), _rspec(), _rspec(), _wspec(),
                    _wspec(), _wspec(), _wspec(), _wspec(), _bspec(),
                    _bspec()],
                   (s2.reshape(2, NR, 128), krinv, sub4, ef4,
                    p[0], p[1], p[2], p[3], B31b, p[4], bias_prev))
        xs4.append(x)
        s2 = edge_k(M.reshape(2 * NPAD, D32), ptr, dste, z32)

    bias4 = conv_w(params['conv4'])[5]
    fW1, fb1 = params['fc1']
    fW2, fb2 = params['fc2']
    fW3, fb3 = params['fc3']
    fW4, fb4 = params['fc4']
    Wk = [_bd(fW1[32 * k:32 * (k + 1), :]) for k in range(5)]
    W4b = _bd(jnp.broadcast_to(fW4, (D32, D32)))
    b4t = jnp.full((8, 128), fb4[0], f32)

    y4 = _tc(_f_body, _rspec(), x_struct,
             [_sspec()] + [_rspec()] * 6 + [_wspec()] * 5 +
             [_bspec(), _bspec(), _wspec(), _bspec(), _wspec(), _bspec(),
              _wspec(), _bspec()],
             (s2.reshape(2, NR, 128), krinv, sub4, xs4[0], xs4[1], xs4[2],
              xs4[3], Wk[0], Wk[1], Wk[2], Wk[3], Wk[4], bias4, _btile(fb1),
              _bd(_pad_w(fW2)), _btile(fb2), _bd(_pad_w(fW3)), _btile(fb3),
              W4b, b4t))

    gpad = 30720 - NVK
    gidx = jnp.concatenate([av, jnp.arange(gpad, dtype=jnp.int32) % 8192])
    yg = _sc_gather(NPAD, D32, 10, 96)(y4.reshape(NPAD, D32),
                                       gidx.reshape(NW, 10, 96))
    return yg[:NVK, 0]
